# baseline (speedup 1.0000x reference)
import functools
import math

import jax
import jax.numpy as jnp
from jax import lax
from jax.experimental import pallas as pl
from jax.experimental.pallas import tpu as pltpu
from jax.experimental.pallas import tpu_sc as plsc

_L = 16
_CHUNK = 400


@functools.lru_cache(maxsize=None)
def _make_sc_kernel(E, N):
    info = plsc.get_sparse_core_info()
    nc, ns = info.num_cores, info.num_subcores
    nw = nc * ns
    assert E % (nw * _CHUNK) == 0
    edges_per_tile = E // nw
    n_chunks = edges_per_tile // _CHUNK
    n2 = 2 * N
    wpt = -(-(n2 // ns) // _L) * _L
    n2p = ns * wpt

    mesh = plsc.VectorSubcoreMesh(core_axis_name="c", subcore_axis_name="s")

    lin_t = [
        pltpu.VMEM((_CHUNK,), jnp.int32),
        pltpu.VMEM((_CHUNK,), jnp.int32),
        pltpu.VMEM((_CHUNK,), jnp.float32),
        pltpu.VMEM((_CHUNK,), jnp.float32),
        pltpu.VMEM((_CHUNK,), jnp.float32),
    ]
    row_t = [
        pltpu.VMEM((_CHUNK, 8), jnp.float32),
        pltpu.VMEM((_CHUNK, 8), jnp.float32),
    ]

    @functools.partial(
        pl.kernel,
        mesh=mesh,
        compiler_params=pltpu.CompilerParams(
            needs_layout_passes=False, use_tc_tiling_on_sc=False),
        out_type=jax.ShapeDtypeStruct((nw * n2,), jnp.float32),
        scratch_types=(
            lin_t + lin_t + row_t + row_t
            + [
                pltpu.VMEM((_L,), jnp.float32),
                pltpu.VMEM((n2,), jnp.float32),
                pltpu.SemaphoreType.DMA,
                pltpu.SemaphoreType.DMA,
                pltpu.SemaphoreType.DMA,
                pltpu.SemaphoreType.DMA,
            ]
        ),
    )
    def k(ta_hbm, tb_hbm, i_hbm, j_hbm, dx_hbm, dy_hbm, q_hbm, eps_hbm,
          out_hbm,
          i0, j0, dx0, dy0, q0, i1, j1, dx1, dy1, q1,
          ta0, tb0, ta1, tb1, eps_v, acc,
          sl0, sl1, sg0, sg1):
        cid = lax.axis_index("c")
        sid = lax.axis_index("s")
        wid = sid * nc + cid
        lin = [(i0, j0, dx0, dy0, q0), (i1, j1, dx1, dy1, q1)]
        rows_b = [(ta0, tb0), (ta1, tb1)]
        sem_l = [sl0, sl1]
        sem_g = [sg0, sg1]

        @plsc.parallel_loop(0, n2 // _L, unroll=8)
        def _zero(g):
            acc[pl.ds(g * _L, _L)] = jnp.zeros((_L,), jnp.float32)

        pltpu.sync_copy(eps_hbm, eps_v)
        eps = eps_v[...]
        iota = lax.iota(jnp.int32, _L)
        cols = [jnp.full((_L,), c, jnp.int32) for c in range(8)]

        def lin_pairs(ci, s):
            base = wid * edges_per_tile + ci * _CHUNK
            bufs = lin[s]
            srcs = (i_hbm, j_hbm, dx_hbm, dy_hbm, q_hbm)
            return [(h.at[pl.ds(base, _CHUNK)], b)
                    for h, b in zip(srcs, bufs)]

        def issue_lin(ci, s):
            for src, dst in lin_pairs(ci, s):
                pltpu.make_async_copy(src, dst, sem_l[s]).start()

        def drain_lin(ci, s):
            for src, dst in lin_pairs(ci, s):
                pltpu.make_async_copy(src, dst, sem_l[s]).wait()

        def issue_gather(s):
            pltpu.make_async_copy(
                ta_hbm.at[lin[s][0]], rows_b[s][0], sem_g[s]).start()
            pltpu.make_async_copy(
                tb_hbm.at[lin[s][1]], rows_b[s][1], sem_g[s]).start()

        def drain_gather(s):
            pltpu.make_async_copy(
                ta_hbm.at[lin[s][0]], rows_b[s][0], sem_g[s]).wait()
            pltpu.make_async_copy(
                tb_hbm.at[lin[s][1]], rows_b[s][1], sem_g[s]).wait()

        def compute(s):
            ta_v, tb_v = rows_b[s]
            i_v, _, dx_v, dy_v, q_v = lin[s]

            @plsc.parallel_loop(0, _CHUNK // _L, unroll=4)
            def group(g):
                b = g * _L
                rws = b + iota

                def col(ref, c):
                    return plsc.load_gather(ref, [rws, cols[c]])

                rjx, rjy = col(ta_v, 0), col(ta_v, 1)
                uix, uiy = col(ta_v, 2), col(ta_v, 3)
                mu, rhi = col(ta_v, 4), col(ta_v, 5)
                rix, riy = col(tb_v, 0), col(tb_v, 1)
                ujx, ujy = col(tb_v, 2), col(tb_v, 3)
                mjc, rhj = col(tb_v, 4), col(tb_v, 5)
                dx = dx_v[pl.ds(b, _L)]
                dy = dy_v[pl.ds(b, _L)]
                q = q_v[pl.ds(b, _L)]

                rx = rix - rjx
                ry = riy - rjy
                rij2 = rx * rx + ry * ry + eps
                dotd = dx * rx + dy * ry
                omq = 1.0 - q
                w = q * omq * omq * omq
                nom = mjc * mu * w * dotd
                rs = rhi + rhj
                den = rs * rs * rij2
                s_ = -(nom / den)
                cx = s_ * (uix - ujx)
                cy = s_ * (uiy - ujy)
                iv = i_v[pl.ds(b, _L)]
                iv2 = iv + iv
                plsc.addupdate_scatter(acc, [iv2], cx)
                plsc.addupdate_scatter(acc, [iv2 + 1], cy)

        issue_lin(0, 0)
        issue_lin(1, 1)
        drain_lin(0, 0)
        issue_gather(0)

        def body(kk, carry):
            for par in (0, 1):
                ci = 2 * kk + par

                @pl.when(ci < n_chunks)
                def _chunk():
                    drain_gather(par)
                    compute(par)

                    @pl.when(ci + 1 < n_chunks)
                    def _next():
                        drain_lin(ci + 1, 1 - par)
                        issue_gather(1 - par)

                    @pl.when(ci + 2 < n_chunks)
                    def _next2():
                        issue_lin(ci + 2, par)
            return carry

        lax.fori_loop(0, (n_chunks + 1) // 2, body, 0)
        pltpu.sync_copy(acc, out_hbm.at[pl.ds(wid * n2, n2)])

    return k


def kernel(i, j, ri, rj, Vi, Vj, distances, radialDistances, support,
           numParticles, eps, rhoi, rhoj, ui, uj, alpha, c0, restDensity):
    E = i.shape[0]
    N = ri.shape[0]
    sup = jnp.asarray(support, jnp.float32)
    cdw = jnp.float32(-20.0 * 7.0 / math.pi) / (sup * sup * sup)
    mu = jnp.asarray(alpha, jnp.float32) * (rhoi + rhoj)
    mjc = 4.0 * cdw * rhoj * Vj
    pad = jnp.zeros((N, 2), jnp.float32)
    ta = jnp.concatenate(
        [rj, ui, mu[:, None], rhoi[:, None], pad], axis=1)
    tb = jnp.concatenate(
        [ri, uj, mjc[:, None], rhoj[:, None], pad], axis=1)
    eps_b = jnp.full((_L,), jnp.asarray(eps, jnp.float32))
    parts = _make_sc_kernel(E, N)(
        ta, tb, i.astype(jnp.int32), j.astype(jnp.int32),
        distances[:, 0], distances[:, 1],
        radialDistances.astype(jnp.float32), eps_b)
    red = jnp.dot(jnp.ones((32,), jnp.float32), parts.reshape(32, 2 * N),
                  preferred_element_type=jnp.float32)
    return red.reshape(N, 2)


# speedup vs baseline: 1.1202x; 1.1202x over previous
import functools
import math

import jax
import jax.numpy as jnp
from jax import lax
from jax.experimental import pallas as pl
from jax.experimental.pallas import tpu as pltpu
from jax.experimental.pallas import tpu_sc as plsc

_L = 16
_CHUNK = 400


@functools.lru_cache(maxsize=None)
def _make_sc_kernel(E, N):
    info = plsc.get_sparse_core_info()
    nc, ns = info.num_cores, info.num_subcores
    nw = nc * ns
    assert E % (nw * _CHUNK) == 0
    edges_per_tile = E // nw
    n_chunks = edges_per_tile // _CHUNK
    n2 = 2 * N
    blk = 5 * _CHUNK

    mesh = plsc.VectorSubcoreMesh(core_axis_name="c", subcore_axis_name="s")

    @functools.partial(
        pl.kernel,
        mesh=mesh,
        compiler_params=pltpu.CompilerParams(
            needs_layout_passes=False, use_tc_tiling_on_sc=False),
        out_type=jax.ShapeDtypeStruct((nw * n2,), jnp.float32),
        scratch_types=[
            pltpu.VMEM((blk,), jnp.int32),
            pltpu.VMEM((blk,), jnp.int32),
            pltpu.VMEM((_CHUNK, 8), jnp.float32),
            pltpu.VMEM((_CHUNK, 8), jnp.float32),
            pltpu.VMEM((_CHUNK, 8), jnp.float32),
            pltpu.VMEM((_CHUNK, 8), jnp.float32),
            pltpu.VMEM((_L,), jnp.float32),
            pltpu.VMEM((n2,), jnp.float32),
            pltpu.SemaphoreType.DMA,
            pltpu.SemaphoreType.DMA,
            pltpu.SemaphoreType.DMA,
            pltpu.SemaphoreType.DMA,
        ],
    )
    def k(ta_hbm, tb_hbm, ed_hbm, eps_hbm, out_hbm,
          ed0, ed1, ta0, tb0, ta1, tb1, eps_v, acc,
          se0, se1, sg0, sg1):
        cid = lax.axis_index("c")
        sid = lax.axis_index("s")
        wid = sid * nc + cid
        ed = [ed0, ed1]
        rows_b = [(ta0, tb0), (ta1, tb1)]
        sem_e = [se0, se1]
        sem_g = [sg0, sg1]

        def ed_pair(ci, s):
            base = (wid * n_chunks + ci) * blk
            return ed_hbm.at[pl.ds(base, blk)], ed[s]

        def issue_ed(ci, s):
            src, dst = ed_pair(ci, s)
            pltpu.make_async_copy(src, dst, sem_e[s]).start()

        def drain_ed(ci, s):
            src, dst = ed_pair(ci, s)
            pltpu.make_async_copy(src, dst, sem_e[s]).wait()

        def issue_gather(s):
            pltpu.make_async_copy(
                ta_hbm.at[ed[s].at[pl.ds(0, _CHUNK)]],
                rows_b[s][0], sem_g[s]).start()
            pltpu.make_async_copy(
                tb_hbm.at[ed[s].at[pl.ds(_CHUNK, _CHUNK)]],
                rows_b[s][1], sem_g[s]).start()

        def drain_gather(s):
            pltpu.make_async_copy(
                ta_hbm.at[ed[s].at[pl.ds(0, _CHUNK)]],
                rows_b[s][0], sem_g[s]).wait()
            pltpu.make_async_copy(
                tb_hbm.at[ed[s].at[pl.ds(_CHUNK, _CHUNK)]],
                rows_b[s][1], sem_g[s]).wait()

        issue_ed(0, 0)
        issue_ed(1, 1)

        @plsc.parallel_loop(0, n2 // _L, unroll=8)
        def _zero(g):
            acc[pl.ds(g * _L, _L)] = jnp.zeros((_L,), jnp.float32)

        pltpu.sync_copy(eps_hbm, eps_v)
        eps = eps_v[...]
        iota = lax.iota(jnp.int32, _L)
        cols = [jnp.full((_L,), c, jnp.int32) for c in range(8)]

        def compute(s):
            ta_v, tb_v = rows_b[s]
            ed_v = ed[s]

            @plsc.parallel_loop(0, _CHUNK // _L, unroll=4)
            def group(g):
                b = g * _L
                rws = b + iota

                def col(ref, c):
                    return plsc.load_gather(ref, [rws, cols[c]])

                rjx, rjy = col(ta_v, 0), col(ta_v, 1)
                uix, uiy = col(ta_v, 2), col(ta_v, 3)
                mu, rhi = col(ta_v, 4), col(ta_v, 5)
                rix, riy = col(tb_v, 0), col(tb_v, 1)
                ujx, ujy = col(tb_v, 2), col(tb_v, 3)
                mjc, rhj = col(tb_v, 4), col(tb_v, 5)
                dx = plsc.bitcast(ed_v[pl.ds(2 * _CHUNK + b, _L)],
                                  jnp.float32)
                dy = plsc.bitcast(ed_v[pl.ds(3 * _CHUNK + b, _L)],
                                  jnp.float32)
                q = plsc.bitcast(ed_v[pl.ds(4 * _CHUNK + b, _L)],
                                 jnp.float32)

                rx = rix - rjx
                ry = riy - rjy
                rij2 = rx * rx + ry * ry + eps
                dotd = dx * rx + dy * ry
                omq = 1.0 - q
                w = q * omq * omq * omq
                nom = mjc * mu * w * dotd
                rs = rhi + rhj
                den = rs * rs * rij2
                s_ = -(nom / den)
                cx = s_ * (uix - ujx)
                cy = s_ * (uiy - ujy)
                iv = ed_v[pl.ds(b, _L)]
                iv2 = iv + iv
                plsc.addupdate_scatter(acc, [iv2], cx)
                plsc.addupdate_scatter(acc, [iv2 + 1], cy)

        drain_ed(0, 0)
        issue_gather(0)

        def body(kk, carry):
            for par in (0, 1):
                ci = 2 * kk + par

                @pl.when(ci < n_chunks)
                def _chunk():
                    drain_gather(par)

                    @pl.when(ci + 1 < n_chunks)
                    def _next():
                        drain_ed(ci + 1, 1 - par)
                        issue_gather(1 - par)

                    compute(par)

                    @pl.when(ci + 2 < n_chunks)
                    def _next2():
                        issue_ed(ci + 2, par)
            return carry

        lax.fori_loop(0, (n_chunks + 1) // 2, body, 0)
        pltpu.sync_copy(acc, out_hbm.at[pl.ds(wid * n2, n2)])

    return k


def kernel(i, j, ri, rj, Vi, Vj, distances, radialDistances, support,
           numParticles, eps, rhoi, rhoj, ui, uj, alpha, c0, restDensity):
    E = i.shape[0]
    N = ri.shape[0]
    sup = jnp.asarray(support, jnp.float32)
    cdw = jnp.float32(-20.0 * 7.0 / math.pi) / (sup * sup * sup)
    mu = jnp.asarray(alpha, jnp.float32) * (rhoi + rhoj)
    mjc = 4.0 * cdw * rhoj * Vj
    pad = jnp.zeros((N, 2), jnp.float32)
    ta = jnp.concatenate(
        [rj, ui, mu[:, None], rhoi[:, None], pad], axis=1)
    tb = jnp.concatenate(
        [ri, uj, mjc[:, None], rhoj[:, None], pad], axis=1)
    eps_b = jnp.full((_L,), jnp.asarray(eps, jnp.float32))
    f2i = lambda x: lax.bitcast_convert_type(
        x.astype(jnp.float32), jnp.int32)
    edp = jnp.stack(
        [i.astype(jnp.int32), j.astype(jnp.int32),
         f2i(distances[:, 0]), f2i(distances[:, 1]),
         f2i(radialDistances)], axis=0)
    edp = edp.reshape(5, E // _CHUNK, _CHUNK).transpose(1, 0, 2).reshape(-1)
    parts = _make_sc_kernel(E, N)(ta, tb, edp, eps_b)
    return parts.reshape(32, 2 * N).sum(axis=0).reshape(N, 2)


# speedup vs baseline: 1.6641x; 1.4855x over previous
import functools
import math

import jax
import jax.numpy as jnp
from jax import lax
from jax.experimental import pallas as pl
from jax.experimental.pallas import tpu as pltpu
from jax.experimental.pallas import tpu_sc as plsc

_L = 16
_CHUNK = 400


@functools.lru_cache(maxsize=None)
def _make_sc_kernel(E, N):
    info = plsc.get_sparse_core_info()
    nc, ns = info.num_cores, info.num_subcores
    nw = nc * ns
    assert E % (nw * _CHUNK) == 0
    edges_per_tile = E // nw
    n_chunks = edges_per_tile // _CHUNK
    n2 = 2 * N

    mesh = plsc.VectorSubcoreMesh(core_axis_name="c", subcore_axis_name="s")

    lin_t = [
        pltpu.VMEM((_CHUNK,), jnp.int32),
        pltpu.VMEM((_CHUNK,), jnp.int32),
        pltpu.VMEM((_CHUNK,), jnp.float32),
        pltpu.VMEM((_CHUNK,), jnp.float32),
        pltpu.VMEM((_CHUNK,), jnp.float32),
    ]
    row_t = [
        pltpu.VMEM((_CHUNK, 8), jnp.float32),
        pltpu.VMEM((_CHUNK, 8), jnp.float32),
    ]

    @functools.partial(
        pl.kernel,
        mesh=mesh,
        compiler_params=pltpu.CompilerParams(
            needs_layout_passes=False, use_tc_tiling_on_sc=False),
        out_type=jax.ShapeDtypeStruct((nw * n2,), jnp.float32),
        scratch_types=(
            lin_t + lin_t + row_t + row_t
            + [
                pltpu.VMEM((_L,), jnp.float32),
                pltpu.VMEM((n2,), jnp.float32),
                pltpu.SemaphoreType.DMA,
                pltpu.SemaphoreType.DMA,
                pltpu.SemaphoreType.DMA,
                pltpu.SemaphoreType.DMA,
            ]
        ),
    )
    def k(ta_hbm, tb_hbm, i_hbm, j_hbm, dx_hbm, dy_hbm, q_hbm, eps_hbm,
          out_hbm,
          i0, j0, dx0, dy0, q0, i1, j1, dx1, dy1, q1,
          ta0, tb0, ta1, tb1, eps_v, acc,
          sl0, sl1, sg0, sg1):
        cid = lax.axis_index("c")
        sid = lax.axis_index("s")
        wid = sid * nc + cid
        lin = [(i0, j0, dx0, dy0, q0), (i1, j1, dx1, dy1, q1)]
        rows_b = [(ta0, tb0), (ta1, tb1)]
        sem_l = [sl0, sl1]
        sem_g = [sg0, sg1]

        def lin_pairs(ci, s):
            base = wid * edges_per_tile + ci * _CHUNK
            srcs = (i_hbm, j_hbm, dx_hbm, dy_hbm, q_hbm)
            return [(h.at[pl.ds(base, _CHUNK)], b)
                    for h, b in zip(srcs, lin[s])]

        def issue_lin(ci, s):
            for src, dst in lin_pairs(ci, s):
                pltpu.make_async_copy(src, dst, sem_l[s]).start()

        def drain_lin(ci, s):
            for src, dst in lin_pairs(ci, s):
                pltpu.make_async_copy(src, dst, sem_l[s]).wait()

        def issue_gather(s):
            pltpu.make_async_copy(
                ta_hbm.at[lin[s][0]], rows_b[s][0], sem_g[s]).start()
            pltpu.make_async_copy(
                tb_hbm.at[lin[s][1]], rows_b[s][1], sem_g[s]).start()

        def drain_gather(s):
            pltpu.make_async_copy(
                ta_hbm.at[lin[s][0]], rows_b[s][0], sem_g[s]).wait()
            pltpu.make_async_copy(
                tb_hbm.at[lin[s][1]], rows_b[s][1], sem_g[s]).wait()

        issue_lin(0, 0)
        issue_lin(1, 1)

        @plsc.parallel_loop(0, n2 // _L, unroll=8)
        def _zero(g):
            acc[pl.ds(g * _L, _L)] = jnp.zeros((_L,), jnp.float32)

        pltpu.sync_copy(eps_hbm, eps_v)
        eps = eps_v[...]
        iota = lax.iota(jnp.int32, _L)
        cols = [jnp.full((_L,), c, jnp.int32) for c in range(8)]

        def compute(s):
            ta_v, tb_v = rows_b[s]
            i_v, _, dx_v, dy_v, q_v = lin[s]

            @plsc.parallel_loop(0, _CHUNK // _L, unroll=4)
            def group(g):
                b = g * _L
                rws = b + iota

                def col(ref, c):
                    return plsc.load_gather(ref, [rws, cols[c]])

                rjx, rjy = col(ta_v, 0), col(ta_v, 1)
                uix, uiy = col(ta_v, 2), col(ta_v, 3)
                mu, rhi = col(ta_v, 4), col(ta_v, 5)
                rix, riy = col(tb_v, 0), col(tb_v, 1)
                ujx, ujy = col(tb_v, 2), col(tb_v, 3)
                mjc, rhj = col(tb_v, 4), col(tb_v, 5)
                dx = dx_v[pl.ds(b, _L)]
                dy = dy_v[pl.ds(b, _L)]
                q = q_v[pl.ds(b, _L)]

                rx = rix - rjx
                ry = riy - rjy
                rij2 = rx * rx + ry * ry + eps
                dotd = dx * rx + dy * ry
                omq = 1.0 - q
                w = q * omq * omq * omq
                nom = mjc * mu * w * dotd
                rs = rhi + rhj
                den = rs * rs * rij2
                s_ = -(nom / den)
                cx = s_ * (uix - ujx)
                cy = s_ * (uiy - ujy)
                iv = i_v[pl.ds(b, _L)]
                iv2 = iv + iv
                plsc.addupdate_scatter(acc, [iv2], cx)
                plsc.addupdate_scatter(acc, [iv2 + 1], cy)

        drain_lin(0, 0)
        issue_gather(0)

        def body(kk, carry):
            for par in (0, 1):
                ci = 2 * kk + par

                @pl.when(ci < n_chunks)
                def _chunk():
                    drain_gather(par)

                    @pl.when(ci + 1 < n_chunks)
                    def _next():
                        drain_lin(ci + 1, 1 - par)
                        issue_gather(1 - par)

                    compute(par)

                    @pl.when(ci + 2 < n_chunks)
                    def _next2():
                        issue_lin(ci + 2, par)
            return carry

        lax.fori_loop(0, (n_chunks + 1) // 2, body, 0)
        pltpu.sync_copy(acc, out_hbm.at[pl.ds(wid * n2, n2)])

    return k


def kernel(i, j, ri, rj, Vi, Vj, distances, radialDistances, support,
           numParticles, eps, rhoi, rhoj, ui, uj, alpha, c0, restDensity):
    E = i.shape[0]
    N = ri.shape[0]
    sup = jnp.asarray(support, jnp.float32)
    cdw = jnp.float32(-20.0 * 7.0 / math.pi) / (sup * sup * sup)
    mu = jnp.asarray(alpha, jnp.float32) * (rhoi + rhoj)
    mjc = 4.0 * cdw * rhoj * Vj
    pad = jnp.zeros((N, 2), jnp.float32)
    ta = jnp.concatenate(
        [rj, ui, mu[:, None], rhoi[:, None], pad], axis=1)
    tb = jnp.concatenate(
        [ri, uj, mjc[:, None], rhoj[:, None], pad], axis=1)
    eps_b = jnp.full((_L,), jnp.asarray(eps, jnp.float32))
    parts = _make_sc_kernel(E, N)(
        ta, tb, i.astype(jnp.int32), j.astype(jnp.int32),
        distances[:, 0], distances[:, 1],
        radialDistances.astype(jnp.float32), eps_b)
    return parts.reshape(32, 2 * N).sum(axis=0).reshape(N, 2)
